# baseline (speedup 1.0000x reference)
import functools
import math

import jax
import jax.numpy as jnp
from jax import lax
from jax.experimental import pallas as pl
from jax.experimental.pallas import tpu as pltpu
from jax.experimental.pallas import tpu_sc as plsc

NUM_CORES = 2
NUM_SUBCORES = 16
NUM_WORKERS = NUM_CORES * NUM_SUBCORES
CHUNK = 1024
SUB = CHUNK // 4


def kernel(token_ids, W):
    B, S = token_ids.shape
    V, D = W.shape
    N = B * S
    scale = math.sqrt(D)
    n_per_w = N // NUM_WORKERS
    n_chunks = n_per_w // CHUNK
    assert n_chunks * CHUNK * NUM_WORKERS == N

    idx = token_ids.reshape(N).astype(jnp.int32)
    idxp = idx.reshape(N // CHUNK, SUB, 4).transpose(0, 2, 1).reshape(N)
    mesh = plsc.VectorSubcoreMesh(core_axis_name="c", subcore_axis_name="s")

    @functools.partial(
        pl.kernel,
        mesh=mesh,
        compiler_params=pltpu.CompilerParams(use_tc_tiling_on_sc=False),
        out_type=jax.ShapeDtypeStruct((N * D // 128, 128), jnp.float32),
        scratch_types=[
            pltpu.VMEM((CHUNK,), jnp.int32),
            pltpu.VMEM((SUB, D), jnp.float32),
            pltpu.SemaphoreType.DMA,
        ],
    )
    def emb(idx_hbm, w_hbm, out_hbm, idx_v, rows_v, sem):
        wid = lax.axis_index("s") * NUM_CORES + lax.axis_index("c")
        base = wid * n_per_w

        @pl.loop(0, n_chunks)
        def _(ci):
            cb = base + ci * CHUNK
            pltpu.sync_copy(idx_hbm.at[pl.ds(cb, CHUNK)], idx_v)
            for k in range(4):
                pltpu.async_copy(
                    w_hbm.at[idx_v.at[pl.ds(k * SUB, SUB)]], rows_v, sem
                ).wait()

                @pl.loop(0, SUB)
                def _(r):
                    for c in range(0, D, 16):
                        sl = (r, pl.ds(c, 16))
                        rows_v.at[sl][...] = rows_v.at[sl][...] * scale

                pltpu.sync_copy(
                    rows_v,
                    out_hbm.at[pl.ds(cb * D // 128, SUB), pl.ds(k * D, D)],
                )

    stage = emb(idxp, W)
    return stage.reshape(B, S, D)


# speedup vs baseline: 1.2004x; 1.2004x over previous
import functools
import math

import jax
import jax.numpy as jnp
from jax import lax
from jax.experimental import pallas as pl
from jax.experimental.pallas import tpu as pltpu
from jax.experimental.pallas import tpu_sc as plsc

NUM_CORES = 2
NUM_SUBCORES = 16
NUM_WORKERS = NUM_CORES * NUM_SUBCORES
CHUNK = 800
PARTS = 4


def _emb_call(n_part, V, D, scale):
    n_per_w = n_part // NUM_WORKERS
    n_chunks = n_per_w // CHUNK
    assert n_chunks * CHUNK * NUM_WORKERS == n_part
    mesh = plsc.VectorSubcoreMesh(core_axis_name="c", subcore_axis_name="s")

    @functools.partial(
        pl.kernel,
        mesh=mesh,
        compiler_params=pltpu.CompilerParams(use_tc_tiling_on_sc=False),
        out_type=jax.ShapeDtypeStruct((n_part, D), jnp.float32),
        scratch_types=[
            pltpu.VMEM((CHUNK,), jnp.int32),
            pltpu.VMEM((CHUNK, D), jnp.float32),
            pltpu.SemaphoreType.DMA,
        ],
    )
    def emb(idx_hbm, w_hbm, out_hbm, idx_v, rows_v, sem):
        wid = lax.axis_index("s") * NUM_CORES + lax.axis_index("c")
        base = wid * n_per_w

        @pl.loop(0, n_chunks)
        def _(ci):
            cb = base + ci * CHUNK
            pltpu.sync_copy(idx_hbm.at[pl.ds(cb, CHUNK)], idx_v)
            pltpu.async_copy(w_hbm.at[idx_v], rows_v, sem).wait()

            @pl.loop(0, CHUNK)
            def _(r):
                for c in range(0, D, 16):
                    sl = (r, pl.ds(c, 16))
                    rows_v.at[sl][...] = rows_v.at[sl][...] * scale

            pltpu.sync_copy(rows_v, out_hbm.at[pl.ds(cb, CHUNK)])

    return emb


def kernel(token_ids, W):
    B, S = token_ids.shape
    V, D = W.shape
    N = B * S
    scale = math.sqrt(D)
    n_part = N // PARTS
    b_part = B // PARTS
    assert n_part * PARTS == N and b_part * PARTS == B

    idx = token_ids.reshape(N).astype(jnp.int32)
    emb = _emb_call(n_part, V, D, scale)
    parts = []
    for p in range(PARTS):
        stage = emb(lax.dynamic_slice_in_dim(idx, p * n_part, n_part), W)
        parts.append(stage.reshape(b_part, S, D))
    return jnp.concatenate(parts, axis=0)
